# baseline (speedup 1.0000x reference)
import functools

import jax
import jax.numpy as jnp
from jax import lax
from jax.experimental import pallas as pl
from jax.experimental.pallas import tpu as pltpu
from jax.experimental.pallas import tpu_sc as plsc

N = 16384
K = 1000
NC = 2
NS = 16
NW = NC * NS
RPW = N // NW
C = 64
NCHUNK = RPW // C
L = 16


def _onehot_body(x_hbm, out_hbm, x_v, buf):
    wid = lax.axis_index("s") * NC + lax.axis_index("c")
    base = wid * RPW

    pltpu.sync_copy(x_hbm.at[pl.ds(base, RPW)], x_v)

    zeros = jnp.zeros((L,), jnp.int32)
    ones = jnp.ones((L,), jnp.int32)
    iota = lax.iota(jnp.int32, L)

    def zbody(i, carry):
        off = i * 64
        buf[pl.ds(off, L)] = zeros
        buf[pl.ds(off + 16, L)] = zeros
        buf[pl.ds(off + 32, L)] = zeros
        buf[pl.ds(off + 48, L)] = zeros
        return carry

    lax.fori_loop(0, C * K // 64, zbody, 0)

    for chunk in range(NCHUNK):
        for j in range(C // L):
            cols = x_v[pl.ds(chunk * C + j * L, L)]
            flat = (iota + j * L) * K + cols
            plsc.store_scatter(buf, [flat], ones)
        pltpu.sync_copy(buf, out_hbm.at[pl.ds(base * K + chunk * C * K, C * K)])
        for j in range(C // L):
            cols = x_v[pl.ds(chunk * C + j * L, L)]
            flat = (iota + j * L) * K + cols
            plsc.store_scatter(buf, [flat], zeros)


@jax.jit
def kernel(x):
    run = functools.partial(
        pl.kernel,
        out_type=jax.ShapeDtypeStruct((N * K,), jnp.int32),
        mesh=plsc.VectorSubcoreMesh(core_axis_name="c", subcore_axis_name="s"),
        compiler_params=pltpu.CompilerParams(needs_layout_passes=False),
        scratch_types=[
            pltpu.VMEM((RPW,), jnp.int32),
            pltpu.VMEM((C * K,), jnp.int32),
        ],
    )(_onehot_body)
    return run(x).reshape(N, K)


# speedup vs baseline: 1.6096x; 1.6096x over previous
import functools

import jax
import jax.numpy as jnp
from jax import lax
from jax.experimental import pallas as pl
from jax.experimental.pallas import tpu as pltpu
from jax.experimental.pallas import tpu_sc as plsc

N = 16384
K = 1000
NC = 2
NS = 16
NW = NC * NS
RPW = N // NW
C = 64
NCHUNK = RPW // C
L = 16


def _onehot_body(x_hbm, out_hbm, x_v, buf):
    wid = lax.axis_index("s") * NC + lax.axis_index("c")
    base = wid * RPW

    pltpu.sync_copy(x_hbm.at[pl.ds(base, RPW)], x_v)

    zeros = jnp.zeros((L,), jnp.int32)
    ones = jnp.ones((L,), jnp.int32)
    iota = lax.iota(jnp.int32, L)

    def zbody(r, carry):
        for c0 in range(0, K - L + 1, L):
            buf[r, pl.ds(c0, L)] = zeros
        buf[r, pl.ds(K - L, L)] = zeros
        return carry

    lax.fori_loop(0, C, zbody, 0)

    for chunk in range(NCHUNK):
        for j in range(C // L):
            cols = x_v[pl.ds(chunk * C + j * L, L)]
            rows = iota + j * L
            plsc.store_scatter(buf, [rows, cols], ones)
        pltpu.sync_copy(buf, out_hbm.at[pl.ds(base + chunk * C, C)])
        for j in range(C // L):
            cols = x_v[pl.ds(chunk * C + j * L, L)]
            rows = iota + j * L
            plsc.store_scatter(buf, [rows, cols], zeros)


@jax.jit
def kernel(x):
    run = functools.partial(
        pl.kernel,
        out_type=jax.ShapeDtypeStruct((N, K), jnp.int32),
        mesh=plsc.VectorSubcoreMesh(core_axis_name="c", subcore_axis_name="s"),
        compiler_params=pltpu.CompilerParams(needs_layout_passes=False),
        scratch_types=[
            pltpu.VMEM((RPW,), jnp.int32),
            pltpu.VMEM((C, K), jnp.int32),
        ],
    )(_onehot_body)
    return run(x)


# speedup vs baseline: 2.9211x; 1.8149x over previous
import functools

import jax
import jax.numpy as jnp
from jax import lax
from jax.experimental import pallas as pl
from jax.experimental.pallas import tpu as pltpu
from jax.experimental.pallas import tpu_sc as plsc

N = 16384
K = 1000
NC = 2
NS = 16
NW = NC * NS
SPW = N // NW
C = 128
NCHUNK = SPW // C
L = 16


def _onehot_body(x_hbm, z_hbm, out_hbm, x_v, buf):
    wid = lax.axis_index("s") * NC + lax.axis_index("c")
    base = wid * SPW

    pltpu.sync_copy(x_hbm.at[pl.ds(base, SPW)], x_v)
    pltpu.sync_copy(z_hbm, buf)

    zeros = jnp.zeros((L,), jnp.int32)
    ones = jnp.ones((L,), jnp.int32)
    iota = lax.iota(jnp.int32, L)

    for chunk in range(NCHUNK):
        for j in range(C // L):
            rows = x_v[pl.ds(chunk * C + j * L, L)]
            cols = iota + j * L
            plsc.store_scatter(buf, [rows, cols], ones)
        pltpu.sync_copy(buf, out_hbm.at[:, pl.ds(base + chunk * C, C)])
        if chunk + 1 < NCHUNK:
            for j in range(C // L):
                rows = x_v[pl.ds(chunk * C + j * L, L)]
                cols = iota + j * L
                plsc.store_scatter(buf, [rows, cols], zeros)


@jax.jit
def kernel(x):
    run = functools.partial(
        pl.kernel,
        out_type=jax.ShapeDtypeStruct((K, N), jnp.int32),
        mesh=plsc.VectorSubcoreMesh(core_axis_name="c", subcore_axis_name="s"),
        compiler_params=pltpu.CompilerParams(needs_layout_passes=False),
        scratch_types=[
            pltpu.VMEM((SPW,), jnp.int32),
            pltpu.VMEM((K, C), jnp.int32),
        ],
    )(_onehot_body)
    zeros_chunk = jnp.zeros((K, C), jnp.int32)
    out_t = run(x, zeros_chunk)
    return out_t.T


# speedup vs baseline: 2.9647x; 1.0149x over previous
import functools

import jax
import jax.numpy as jnp
from jax import lax
from jax.experimental import pallas as pl
from jax.experimental.pallas import tpu as pltpu
from jax.experimental.pallas import tpu_sc as plsc

N = 16384
K = 1000
NC = 2
NS = 16
NW = NC * NS
SPW = N // NW
CC = 200
NCHUNK = K // CC
RB = CC // 8
L = 16


def _onehot_body(x_hbm, z_hbm, out_hbm, x_v, buf, sem):
    wid = lax.axis_index("s") * NC + lax.axis_index("c")
    base = wid * SPW

    pltpu.sync_copy(x_hbm.at[pl.ds(base, SPW)], x_v)
    pltpu.sync_copy(z_hbm, buf)

    zeros = jnp.zeros((L,), jnp.int32)
    ones = jnp.ones((L,), jnp.int32)
    iota = lax.iota(jnp.int32, L)

    for chunk in range(NCHUNK):
        c0 = chunk * CC
        for j in range(SPW // L):
            xv = x_v[pl.ds(j * L, L)]
            rows = xv - c0
            mask = (xv >= c0) & (xv < c0 + CC)
            plsc.store_scatter(buf, [rows, iota + j * L], ones, mask=mask)
        copies = [
            pltpu.async_copy(
                buf.at[pl.ds(i * 8, 8), :],
                out_hbm.at[pl.ds(c0 + i * 8, 8), pl.ds(base, SPW)],
                sem,
            )
            for i in range(RB)
        ]
        for cp in copies:
            cp.wait()
        if chunk + 1 < NCHUNK:
            for j in range(SPW // L):
                xv = x_v[pl.ds(j * L, L)]
                rows = xv - c0
                mask = (xv >= c0) & (xv < c0 + CC)
                plsc.store_scatter(buf, [rows, iota + j * L], zeros, mask=mask)


@jax.jit
def kernel(x):
    run = functools.partial(
        pl.kernel,
        out_type=jax.ShapeDtypeStruct((K, N), jnp.int32),
        mesh=plsc.VectorSubcoreMesh(core_axis_name="c", subcore_axis_name="s"),
        compiler_params=pltpu.CompilerParams(needs_layout_passes=False),
        scratch_types=[
            pltpu.VMEM((SPW,), jnp.int32),
            pltpu.VMEM((CC, SPW), jnp.int32),
            pltpu.SemaphoreType.DMA,
        ],
    )(_onehot_body)
    zeros_chunk = jnp.zeros((CC, SPW), jnp.int32)
    out_t = run(x, zeros_chunk)
    return out_t.T
